# baseline (speedup 1.0000x reference)
import numpy as np
import jax
import jax.numpy as jnp
from jax.experimental import pallas as pl
from jax.experimental.pallas import tpu as pltpu

_N, _L, _K = 128, 8192, 20
_C = 2560
_RT = 8
_R = _L // 128
_J = _R // _RT
_G = _C // _K


def _threefry_bits(x1):
    k1 = jnp.uint32(42)
    k2 = jnp.uint32(0 ^ 42 ^ 0x1BD11BDA)
    ks = (jnp.uint32(0), k1, k2)
    rot = ((13, 15, 26, 6), (17, 29, 16, 24))
    x1 = x1 + k1
    x0 = x1
    x1 = ((x1 << 13) | (x1 >> 19)) ^ x0
    for i in range(5):
        rs = rot[i % 2][1:] if i == 0 else rot[i % 2]
        for r in rs:
            x0 = x0 + x1
            x1 = ((x1 << r) | (x1 >> (32 - r))) ^ x0
        x0 = x0 + ks[(i + 1) % 3]
        x1 = x1 + ks[(i + 2) % 3] + jnp.uint32(i + 1)
    return x0 ^ x1


def _body(dt_ref, ct_ref, vc_ref, xt_ref, mk_ref, x_out, c_out,
          pg_ref, pw_ref, e1_ref, e1t_ref):
    n = pl.program_id(0)
    j = pl.program_id(1)

    @pl.when((n == 0) & (j == 0))
    def _init():
        lane = jax.lax.broadcasted_iota(jnp.int32, (_RT, _C), 1)
        pgi = lane % _K
        pg_ref[...] = pgi
        pw_ref[...] = jax.lax.bitcast_convert_type(
            (127 - pgi) << 23, jnp.float32)
        crow = jax.lax.broadcasted_iota(jnp.int32, (_C, _G), 0)
        ccol = jax.lax.broadcasted_iota(jnp.int32, (_C, _G), 1)
        e1_ref[...] = (crow // _K == ccol).astype(jnp.float32)
        trow = jax.lax.broadcasted_iota(jnp.int32, (_G, _C), 0)
        tcol = jax.lax.broadcasted_iota(jnp.int32, (_G, _C), 1)
        e1t_ref[...] = (tcol // _K == trow).astype(jnp.float32)

    ct = ct_ref[0]
    vc = vc_ref[0]
    dtn = dt_ref[n]
    c_new = ct + vc * dtn
    probs = jnp.clip(c_new, 0.0, 1.0) + 1e-8
    v = jnp.log(probs)

    pg = pg_ref[...]
    row = jax.lax.broadcasted_iota(jnp.int32, (_RT, _C), 0)
    lane = jax.lax.broadcasted_iota(jnp.int32, (_RT, _C), 1)
    flat = (n * _R + j * _RT + row) * _C + lane
    bits = _threefry_bits(flat.astype(jnp.uint32))
    fb = (bits >> 9) | jnp.uint32(0x3F800000)
    floats = jax.lax.bitcast_convert_type(fb, jnp.float32) - 1.0
    tiny = jnp.float32(np.finfo(np.float32).tiny)
    u = jnp.maximum(tiny, floats + tiny)
    v = v + (-jnp.log(-jnp.log(u)))

    neg_inf = jnp.float32(-np.inf)
    m = v
    for s in (1, 2, 4, 8, 16):
        cand = pltpu.roll(m, _C - s, 1)
        m = jnp.maximum(m, jnp.where(pg + s < _K, cand, neg_inf))
    for s in (1, 2, 4, 8, 16):
        cand = pltpu.roll(m, s, 1)
        m = jnp.maximum(m, jnp.where(pg - s >= 0, cand, neg_inf))

    contrib = jnp.where(v == m, pw_ref[...], 0.0)
    s2 = jnp.dot(contrib, e1_ref[...],
                 preferred_element_type=jnp.float32)
    xs = 127 - (jax.lax.bitcast_convert_type(s2, jnp.int32) >> 23)

    mk = mk_ref[0]
    x_out[0] = jnp.where(mk != 0, xs, xt_ref[0])

    mke = jnp.dot(mk.astype(jnp.float32), e1t_ref[...],
                  preferred_element_type=jnp.float32)
    c_out[0] = jnp.where(mke > 0.5, c_new, ct)


def kernel(xt, ct, vc_t, dt, mask):
    ct3 = ct.reshape(_N, _R, _C)
    vc3 = vc_t.reshape(_N, _R, _C)
    xt3 = xt.reshape(_N, _R, 128)
    mk3 = mask.astype(jnp.int32).reshape(_N, _R, 128)
    x_new, c_new = pl.pallas_call(
        _body,
        grid=(_N, _J),
        in_specs=[
            pl.BlockSpec(memory_space=pltpu.SMEM),
            pl.BlockSpec((1, _RT, _C), lambda n, j: (n, j, 0)),
            pl.BlockSpec((1, _RT, _C), lambda n, j: (n, j, 0)),
            pl.BlockSpec((1, _RT, 128), lambda n, j: (n, j, 0)),
            pl.BlockSpec((1, _RT, 128), lambda n, j: (n, j, 0)),
        ],
        out_specs=[
            pl.BlockSpec((1, _RT, 128), lambda n, j: (n, j, 0)),
            pl.BlockSpec((1, _RT, _C), lambda n, j: (n, j, 0)),
        ],
        out_shape=[
            jax.ShapeDtypeStruct((_N, _R, 128), jnp.int32),
            jax.ShapeDtypeStruct((_N, _R, _C), jnp.float32),
        ],
        scratch_shapes=[
            pltpu.VMEM((_RT, _C), jnp.int32),
            pltpu.VMEM((_RT, _C), jnp.float32),
            pltpu.VMEM((_C, _G), jnp.float32),
            pltpu.VMEM((_G, _C), jnp.float32),
        ],
    )(dt, ct3, vc3, xt3, mk3)
    return x_new.reshape(_N, _L), c_new.reshape(_N, _L, _K)


# speedup vs baseline: 1.6016x; 1.6016x over previous
import numpy as np
import jax
import jax.numpy as jnp
from jax.experimental import pallas as pl
from jax.experimental.pallas import tpu as pltpu

_N, _L, _K = 128, 8192, 20
_C = 2560
_RT = 64
_R = _L // 128
_J = _R // _RT
_G = _C // _K


def _threefry_bits(x1):
    k1 = jnp.uint32(42)
    k2 = jnp.uint32(0 ^ 42 ^ 0x1BD11BDA)
    ks = (jnp.uint32(0), k1, k2)
    rot = ((13, 15, 26, 6), (17, 29, 16, 24))
    x1 = x1 + k1
    x0 = x1
    x1 = ((x1 << 13) | (x1 >> 19)) ^ x0
    for i in range(5):
        rs = rot[i % 2][1:] if i == 0 else rot[i % 2]
        for r in rs:
            x0 = x0 + x1
            x1 = ((x1 << r) | (x1 >> (32 - r))) ^ x0
        x0 = x0 + ks[(i + 1) % 3]
        x1 = x1 + ks[(i + 2) % 3] + jnp.uint32(i + 1)
    return x0 ^ x1


def _body(dt_ref, ct_ref, vc_ref, xt_ref, mk_ref, x_out, c_out,
          pg_ref, pw_ref, e1_ref, e1t_ref):
    n = pl.program_id(0)
    j = pl.program_id(1)

    @pl.when((n == 0) & (j == 0))
    def _init():
        lane = jax.lax.broadcasted_iota(jnp.int32, (_RT, _C), 1)
        pgi = lane % _K
        pg_ref[...] = pgi
        pw_ref[...] = jax.lax.bitcast_convert_type(
            (127 - pgi) << 23, jnp.float32)
        crow = jax.lax.broadcasted_iota(jnp.int32, (_C, _G), 0)
        ccol = jax.lax.broadcasted_iota(jnp.int32, (_C, _G), 1)
        e1_ref[...] = (crow // _K == ccol).astype(jnp.float32)
        trow = jax.lax.broadcasted_iota(jnp.int32, (_G, _C), 0)
        tcol = jax.lax.broadcasted_iota(jnp.int32, (_G, _C), 1)
        e1t_ref[...] = (tcol // _K == trow).astype(jnp.float32)

    ct = ct_ref[0]
    vc = vc_ref[0]
    dtn = dt_ref[n]
    c_new = ct + vc * dtn
    probs = jnp.clip(c_new, 0.0, 1.0) + 1e-8
    v = jnp.log(probs)

    pg = pg_ref[...]
    row = jax.lax.broadcasted_iota(jnp.int32, (_RT, _C), 0)
    lane = jax.lax.broadcasted_iota(jnp.int32, (_RT, _C), 1)
    flat = (n * _R + j * _RT + row) * _C + lane
    bits = _threefry_bits(flat.astype(jnp.uint32))
    fb = (bits >> 9) | jnp.uint32(0x3F800000)
    floats = jax.lax.bitcast_convert_type(fb, jnp.float32) - 1.0
    tiny = jnp.float32(np.finfo(np.float32).tiny)
    u = jnp.maximum(tiny, floats + tiny)
    v = v + (-jnp.log(-jnp.log(u)))

    neg_inf = jnp.float32(-np.inf)
    m = v
    for s in (1, 2, 4, 8, 16):
        cand = pltpu.roll(m, _C - s, 1)
        m = jnp.maximum(m, jnp.where(pg + s < _K, cand, neg_inf))
    for s in (1, 2, 4, 8, 16):
        cand = pltpu.roll(m, s, 1)
        m = jnp.maximum(m, jnp.where(pg - s >= 0, cand, neg_inf))

    contrib = jnp.where(v == m, pw_ref[...], 0.0)
    s2 = jnp.dot(contrib, e1_ref[...],
                 preferred_element_type=jnp.float32)
    xs = 127 - (jax.lax.bitcast_convert_type(s2, jnp.int32) >> 23)

    mk = mk_ref[0]
    x_out[0] = jnp.where(mk != 0, xs, xt_ref[0])

    mke = jnp.dot(mk.astype(jnp.float32), e1t_ref[...],
                  preferred_element_type=jnp.float32)
    c_out[0] = jnp.where(mke > 0.5, c_new, ct)


def kernel(xt, ct, vc_t, dt, mask):
    ct3 = ct.reshape(_N, _R, _C)
    vc3 = vc_t.reshape(_N, _R, _C)
    xt3 = xt.reshape(_N, _R, 128)
    mk3 = mask.astype(jnp.int32).reshape(_N, _R, 128)
    x_new, c_new = pl.pallas_call(
        _body,
        grid=(_N, _J),
        in_specs=[
            pl.BlockSpec(memory_space=pltpu.SMEM),
            pl.BlockSpec((1, _RT, _C), lambda n, j: (n, j, 0)),
            pl.BlockSpec((1, _RT, _C), lambda n, j: (n, j, 0)),
            pl.BlockSpec((1, _RT, 128), lambda n, j: (n, j, 0)),
            pl.BlockSpec((1, _RT, 128), lambda n, j: (n, j, 0)),
        ],
        out_specs=[
            pl.BlockSpec((1, _RT, 128), lambda n, j: (n, j, 0)),
            pl.BlockSpec((1, _RT, _C), lambda n, j: (n, j, 0)),
        ],
        out_shape=[
            jax.ShapeDtypeStruct((_N, _R, 128), jnp.int32),
            jax.ShapeDtypeStruct((_N, _R, _C), jnp.float32),
        ],
        scratch_shapes=[
            pltpu.VMEM((_RT, _C), jnp.int32),
            pltpu.VMEM((_RT, _C), jnp.float32),
            pltpu.VMEM((_C, _G), jnp.float32),
            pltpu.VMEM((_G, _C), jnp.float32),
        ],
    )(dt, ct3, vc3, xt3, mk3)
    return x_new.reshape(_N, _L), c_new.reshape(_N, _L, _K)


# speedup vs baseline: 3.2809x; 2.0486x over previous
import numpy as np
import jax
import jax.numpy as jnp
from jax.experimental import pallas as pl
from jax.experimental.pallas import tpu as pltpu

_N, _L, _K = 128, 8192, 20
_BN = 8
_BL = 1024


def _threefry_bits(x1):
    k1 = jnp.uint32(42)
    k2 = jnp.uint32(0 ^ 42 ^ 0x1BD11BDA)
    ks = (jnp.uint32(0), k1, k2)
    rot = ((13, 15, 26, 6), (17, 29, 16, 24))
    x0 = x1
    x1 = ((x1 << 13) | (x1 >> 19)) ^ x0
    for i in range(5):
        rs = rot[i % 2][1:] if i == 0 else rot[i % 2]
        for r in rs:
            x0 = x0 + x1
            x1 = ((x1 << r) | (x1 >> (32 - r))) ^ x0
        x0 = x0 + ks[(i + 1) % 3]
        x1 = x1 + ks[(i + 2) % 3] + jnp.uint32(i + 1)
    return x0 ^ x1


def _body(dt_ref, ct_ref, vc_ref, xt_ref, mk_ref, x_out, c_out):
    bn = pl.program_id(0)
    bl = pl.program_id(1)
    ct = ct_ref[...]
    vc = vc_ref[...]
    dtb = dt_ref[:, 0:1][None]
    c_new = ct + vc * dtb
    probs = jnp.clip(c_new, 0.0, 1.0) + 1e-8
    v = jnp.log(probs)

    row = jax.lax.broadcasted_iota(jnp.int32, (_K, _BN, _BL), 1)
    lane = jax.lax.broadcasted_iota(jnp.int32, (_K, _BN, _BL), 2)
    kpl = jax.lax.broadcasted_iota(jnp.int32, (_K, _BN, _BL), 0)
    tok = (bn * _BN + row) * _L + (bl * _BL + lane)
    x1 = (tok * _K + (kpl + 42)).astype(jnp.uint32)
    bits = _threefry_bits(x1)
    fb = (bits >> 9) | jnp.uint32(0x3F800000)
    floats = jax.lax.bitcast_convert_type(fb, jnp.float32) - 1.0
    tiny = jnp.float32(np.finfo(np.float32).tiny)
    u = jnp.maximum(tiny, floats + tiny)
    v = v + (-jnp.log(-jnp.log(u)))

    cur_v = v[0]
    cur_i = jnp.zeros((_BN, _BL), jnp.int32)
    for k in range(1, _K):
        vk = v[k]
        gt = vk > cur_v
        cur_v = jnp.where(gt, vk, cur_v)
        cur_i = jnp.where(gt, k, cur_i)

    mk = mk_ref[...]
    x_out[...] = jnp.where(mk != 0, cur_i, xt_ref[...])
    c_out[...] = jnp.where(mk[None] != 0, c_new, ct)


def kernel(xt, ct, vc_t, dt, mask):
    ctT = jnp.transpose(ct, (2, 0, 1))
    vcT = jnp.transpose(vc_t, (2, 0, 1))
    mk = mask.astype(jnp.int32)
    dtl = jnp.broadcast_to(dt[:, None], (_N, 128))
    x_new, c_newT = pl.pallas_call(
        _body,
        grid=(_N // _BN, _L // _BL),
        in_specs=[
            pl.BlockSpec((_BN, 128), lambda bn, bl: (bn, 0)),
            pl.BlockSpec((_K, _BN, _BL), lambda bn, bl: (0, bn, bl)),
            pl.BlockSpec((_K, _BN, _BL), lambda bn, bl: (0, bn, bl)),
            pl.BlockSpec((_BN, _BL), lambda bn, bl: (bn, bl)),
            pl.BlockSpec((_BN, _BL), lambda bn, bl: (bn, bl)),
        ],
        out_specs=[
            pl.BlockSpec((_BN, _BL), lambda bn, bl: (bn, bl)),
            pl.BlockSpec((_K, _BN, _BL), lambda bn, bl: (0, bn, bl)),
        ],
        out_shape=[
            jax.ShapeDtypeStruct((_N, _L), jnp.int32),
            jax.ShapeDtypeStruct((_K, _N, _L), jnp.float32),
        ],
    )(dtl, ctT, vcT, xt, mk)
    return x_new, jnp.transpose(c_newT, (1, 2, 0))


# speedup vs baseline: 5.0950x; 1.5529x over previous
import numpy as np
import jax
import jax.numpy as jnp
from jax.experimental import pallas as pl
from jax.experimental.pallas import tpu as pltpu

_N, _L, _K = 128, 8192, 20
_BN = 8
_BL = 1024


def _threefry_bits(x1):
    k1 = jnp.uint32(42)
    k2 = jnp.uint32(0 ^ 42 ^ 0x1BD11BDA)
    ks = (jnp.uint32(0), k1, k2)
    rot = ((13, 15, 26, 6), (17, 29, 16, 24))
    x0 = x1
    x1 = ((x1 << 13) | (x1 >> 19)) ^ x0
    for i in range(5):
        rs = rot[i % 2][1:] if i == 0 else rot[i % 2]
        for r in rs:
            x0 = x0 + x1
            x1 = ((x1 << r) | (x1 >> (32 - r))) ^ x0
        x0 = x0 + ks[(i + 1) % 3]
        x1 = x1 + ks[(i + 2) % 3] + jnp.uint32(i + 1)
    return x0 ^ x1


_CH = 128


def _body(dt_ref, ct_ref, vc_ref, xt_ref, mk_ref, x_out, c_out):
    bn = pl.program_id(0)
    bl = pl.program_id(1)
    dtb = dt_ref[:, 0:1][None]
    tiny = jnp.float32(np.finfo(np.float32).tiny)

    row = jax.lax.broadcasted_iota(jnp.int32, (_K, _BN, _CH), 1)
    lane = jax.lax.broadcasted_iota(jnp.int32, (_K, _BN, _CH), 2)
    kpl = jax.lax.broadcasted_iota(jnp.int32, (_K, _BN, _CH), 0)
    tok = (bn * _BN + row) * _L + (bl * _BL + lane)
    x1n = (tok * _K + (kpl + 42)).astype(jnp.uint32)

    for c in range(_BL // _CH):
        sl = slice(c * _CH, (c + 1) * _CH)
        ct = ct_ref[:, :, sl]
        vc = vc_ref[:, :, sl]
        c_new = ct + vc * dtb
        probs = jnp.clip(c_new, 0.0, 1.0) + 1e-8
        v = jnp.log(probs)

        bits = _threefry_bits(x1n)
        x1n = x1n + jnp.uint32(_K * _CH)
        fb = (bits >> 9) | jnp.uint32(0x3F800000)
        floats = jax.lax.bitcast_convert_type(fb, jnp.float32) - 1.0
        u = jnp.maximum(tiny, floats + tiny)
        v = v + (-jnp.log(-jnp.log(u)))

        cur_v = v[0]
        cur_i = jnp.zeros((_BN, _CH), jnp.int32)
        for k in range(1, _K):
            vk = v[k]
            gt = vk > cur_v
            cur_v = jnp.where(gt, vk, cur_v)
            cur_i = jnp.where(gt, k, cur_i)

        mk = mk_ref[:, sl]
        x_out[:, sl] = jnp.where(mk != 0, cur_i, xt_ref[:, sl])
        c_out[:, :, sl] = jnp.where(mk[None] != 0, c_new, ct)


def kernel(xt, ct, vc_t, dt, mask):
    ctT = jnp.transpose(ct, (2, 0, 1))
    vcT = jnp.transpose(vc_t, (2, 0, 1))
    mk = mask.astype(jnp.int32)
    dtl = jnp.broadcast_to(dt[:, None], (_N, 128))
    x_new, c_newT = pl.pallas_call(
        _body,
        grid=(_N // _BN, _L // _BL),
        in_specs=[
            pl.BlockSpec((_BN, 128), lambda bn, bl: (bn, 0)),
            pl.BlockSpec((_K, _BN, _BL), lambda bn, bl: (0, bn, bl)),
            pl.BlockSpec((_K, _BN, _BL), lambda bn, bl: (0, bn, bl)),
            pl.BlockSpec((_BN, _BL), lambda bn, bl: (bn, bl)),
            pl.BlockSpec((_BN, _BL), lambda bn, bl: (bn, bl)),
        ],
        out_specs=[
            pl.BlockSpec((_BN, _BL), lambda bn, bl: (bn, bl)),
            pl.BlockSpec((_K, _BN, _BL), lambda bn, bl: (0, bn, bl)),
        ],
        out_shape=[
            jax.ShapeDtypeStruct((_N, _L), jnp.int32),
            jax.ShapeDtypeStruct((_K, _N, _L), jnp.float32),
        ],
    )(dtl, ctT, vcT, xt, mk)
    return x_new, jnp.transpose(c_newT, (1, 2, 0))


# speedup vs baseline: 5.1097x; 1.0029x over previous
import numpy as np
import jax
import jax.numpy as jnp
from jax.experimental import pallas as pl
from jax.experimental.pallas import tpu as pltpu

_N, _L, _K = 128, 8192, 20
_BN = 8
_BL = 2048


def _threefry_bits(x1):
    k1 = jnp.uint32(42)
    k2 = jnp.uint32(0 ^ 42 ^ 0x1BD11BDA)
    ks = (jnp.uint32(0), k1, k2)
    rot = ((13, 15, 26, 6), (17, 29, 16, 24))
    x0 = x1
    x1 = ((x1 << 13) | (x1 >> 19)) ^ x0
    for i in range(5):
        rs = rot[i % 2][1:] if i == 0 else rot[i % 2]
        for r in rs:
            x0 = x0 + x1
            x1 = ((x1 << r) | (x1 >> (32 - r))) ^ x0
        x0 = x0 + ks[(i + 1) % 3]
        x1 = x1 + ks[(i + 2) % 3] + jnp.uint32(i + 1)
    return x0 ^ x1


_CH = 128


def _body(dt_ref, ct_ref, vc_ref, xt_ref, mk_ref, x_out, c_out):
    bn = pl.program_id(0)
    bl = pl.program_id(1)
    dtb = dt_ref[:, 0:1][None]
    tiny = jnp.float32(np.finfo(np.float32).tiny)

    row = jax.lax.broadcasted_iota(jnp.int32, (_K, _BN, _CH), 1)
    lane = jax.lax.broadcasted_iota(jnp.int32, (_K, _BN, _CH), 2)
    kpl = jax.lax.broadcasted_iota(jnp.int32, (_K, _BN, _CH), 0)
    tok = (bn * _BN + row) * _L + (bl * _BL + lane)
    x1n = (tok * _K + (kpl + 42)).astype(jnp.uint32)

    for c in range(_BL // _CH):
        sl = slice(c * _CH, (c + 1) * _CH)
        ct = ct_ref[:, :, sl]
        vc = vc_ref[:, :, sl]
        c_new = ct + vc * dtb
        probs = jnp.clip(c_new, 0.0, 1.0) + 1e-8
        v = jnp.log(probs)

        bits = _threefry_bits(x1n)
        x1n = x1n + jnp.uint32(_K * _CH)
        fb = (bits >> 9) | jnp.uint32(0x3F800000)
        floats = jax.lax.bitcast_convert_type(fb, jnp.float32) - 1.0
        u = jnp.maximum(tiny, floats + tiny)
        v = v + (-jnp.log(-jnp.log(u)))

        cur_v = v[0]
        cur_i = jnp.zeros((_BN, _CH), jnp.int32)
        for k in range(1, _K):
            vk = v[k]
            gt = vk > cur_v
            cur_v = jnp.where(gt, vk, cur_v)
            cur_i = jnp.where(gt, k, cur_i)

        mk = mk_ref[:, sl]
        x_out[:, sl] = jnp.where(mk != 0, cur_i, xt_ref[:, sl])
        c_out[:, :, sl] = jnp.where(mk[None] != 0, c_new, ct)


def kernel(xt, ct, vc_t, dt, mask):
    ctT = jnp.transpose(ct, (2, 0, 1))
    vcT = jnp.transpose(vc_t, (2, 0, 1))
    mk = mask.astype(jnp.int32)
    dtl = jnp.broadcast_to(dt[:, None], (_N, 128))
    x_new, c_newT = pl.pallas_call(
        _body,
        grid=(_N // _BN, _L // _BL),
        in_specs=[
            pl.BlockSpec((_BN, 128), lambda bn, bl: (bn, 0)),
            pl.BlockSpec((_K, _BN, _BL), lambda bn, bl: (0, bn, bl)),
            pl.BlockSpec((_K, _BN, _BL), lambda bn, bl: (0, bn, bl)),
            pl.BlockSpec((_BN, _BL), lambda bn, bl: (bn, bl)),
            pl.BlockSpec((_BN, _BL), lambda bn, bl: (bn, bl)),
        ],
        out_specs=[
            pl.BlockSpec((_BN, _BL), lambda bn, bl: (bn, bl)),
            pl.BlockSpec((_K, _BN, _BL), lambda bn, bl: (0, bn, bl)),
        ],
        out_shape=[
            jax.ShapeDtypeStruct((_N, _L), jnp.int32),
            jax.ShapeDtypeStruct((_K, _N, _L), jnp.float32),
        ],
    )(dtl, ctT, vcT, xt, mk)
    return x_new, jnp.transpose(c_newT, (1, 2, 0))
